# Initial kernel scaffold; baseline (speedup 1.0000x reference)
import jax
import jax.numpy as jnp
from jax.experimental import pallas as pl


def kernel(xs, gate_w, w1, w2):
    raise NotImplementedError("write your pallas kernel here")


# speedup vs baseline: 1.8041x; 1.8041x over previous
import functools

import jax
import jax.numpy as jnp
from jax import lax
from jax.experimental import pallas as pl
from jax.experimental.pallas import tpu as pltpu
from jax.experimental.pallas import tpu_sc as plsc

IDIM = 1024
HID = 4096
NE = 8
T = 8192
BLK = 256
NROWS = 18432
NBLK = NROWS // BLK

NW = 32
CHUNK = T // NW
DSUB = 32
CSUB = 16

def _sc_mesh():
    return plsc.VectorSubcoreMesh(core_axis_name="c", subcore_axis_name="s")


def _router_body(x_ref, gw_ref, out_ref):
    out_ref[...] = lax.dot_general(
        x_ref[...], gw_ref[...], (((1,), (1,)), ((), ())),
        preferred_element_type=jnp.float32)


def _router(x, gate_w):
    return pl.pallas_call(
        _router_body,
        grid=(8,),
        in_specs=[
            pl.BlockSpec((T // 8, IDIM), lambda b: (b, 0)),
            pl.BlockSpec((NE, IDIM), lambda b: (0, 0)),
        ],
        out_specs=pl.BlockSpec((T // 8, NE), lambda b: (b, 0)),
        out_shape=jax.ShapeDtypeStruct((T, NE), jnp.float32),
    )(x, gate_w)


def _route_meta_body(lg_ref, d0_ref, d1_ref, w0_ref, w1_ref, ps_ref):
    logits = lg_ref[...]
    col = lax.broadcasted_iota(jnp.int32, (T, NE), 1)
    m1 = jnp.max(logits, axis=1, keepdims=True)
    e0 = jnp.min(jnp.where(logits == m1, col, NE), axis=1)
    mask0 = col == e0[:, None]
    l2 = jnp.where(mask0, -jnp.inf, logits)
    m2 = jnp.max(l2, axis=1, keepdims=True)
    e1 = jnp.min(jnp.where(l2 == m2, col, NE), axis=1)
    mask1 = col == e1[:, None]

    w0 = 1.0 / (1.0 + jnp.exp(m2[:, 0] - m1[:, 0]))
    w0_ref[...] = w0
    w1_ref[...] = 1.0 - w0

    s = mask0.astype(jnp.int32) + mask1.astype(jnp.int32)
    c = s
    k = 1
    while k < T:
        c = c + jnp.concatenate([jnp.zeros((k, NE), jnp.int32), c[:-k]], axis=0)
        k *= 2
    excl = c - s

    tot = lax.slice(c, (T - 1, 0), (T, NE))
    pc = ((tot + (BLK - 1)) // BLK) * BLK
    ps = pc
    k = 1
    while k < NE:
        ps = ps + jnp.concatenate([jnp.zeros((1, k), jnp.int32), ps[:, :-k]], axis=1)
        k *= 2
    ps = ps - pc
    ps_ref[...] = ps

    d0_ref[...] = jnp.sum(jnp.where(mask0, excl + ps, 0), axis=1)
    d1_ref[...] = jnp.sum(jnp.where(mask1, excl + ps, 0), axis=1)


def _route_meta(logits):
    return pl.pallas_call(
        _route_meta_body,
        out_shape=(
            jax.ShapeDtypeStruct((T,), jnp.int32),
            jax.ShapeDtypeStruct((T,), jnp.int32),
            jax.ShapeDtypeStruct((T,), jnp.float32),
            jax.ShapeDtypeStruct((T,), jnp.float32),
            jax.ShapeDtypeStruct((1, NE), jnp.int32),
        ),
    )(logits)


def _dispatch_body(x_hbm, d0_hbm, d1_hbm, xs_hbm, xv, i0, i1, sem):
    wid = lax.axis_index("s") * 2 + lax.axis_index("c")
    base = wid * CHUNK

    def step(j, _):
        off = base + j * DSUB
        pltpu.sync_copy(d0_hbm.at[pl.ds(off, DSUB)], i0)
        pltpu.sync_copy(d1_hbm.at[pl.ds(off, DSUB)], i1)
        pltpu.sync_copy(x_hbm.at[pl.ds(off, DSUB)], xv)
        pltpu.async_copy(xv, xs_hbm.at[i0], sem).wait()
        pltpu.async_copy(xv, xs_hbm.at[i1], sem).wait()
        return 0

    lax.fori_loop(0, CHUNK // DSUB, step, 0)


def _dispatch(x, d0, d1):
    f = pl.kernel(
        _dispatch_body,
        out_type=jax.ShapeDtypeStruct((NROWS, IDIM), jnp.float32),
        mesh=_sc_mesh(),
        scratch_types=[
            pltpu.VMEM((DSUB, IDIM), jnp.float32),
            pltpu.VMEM((DSUB,), jnp.int32),
            pltpu.VMEM((DSUB,), jnp.int32),
            pltpu.SemaphoreType.DMA,
        ],
    )
    return f(x, d0, d1)


def _gmm_body(be_ref, xs_ref, w1_ref, w2_ref, out_ref):
    xb = xs_ref[...].astype(jnp.bfloat16)
    h = lax.dot_general(xb, w1_ref[0], (((1,), (1,)), ((), ())),
                        preferred_element_type=jnp.float32)
    h = jnp.maximum(h, 0.0).astype(jnp.bfloat16)
    out_ref[...] = lax.dot_general(h, w2_ref[0], (((1,), (1,)), ((), ())),
                                   preferred_element_type=jnp.float32)


def _gmm(be, xsorted, w1b, w2b):
    return pl.pallas_call(
        _gmm_body,
        grid_spec=pltpu.PrefetchScalarGridSpec(
            num_scalar_prefetch=1,
            grid=(NBLK,),
            in_specs=[
                pl.BlockSpec((BLK, IDIM), lambda b, be_ref: (b, 0)),
                pl.BlockSpec((1, HID, IDIM), lambda b, be_ref: (be_ref[b], 0, 0)),
                pl.BlockSpec((1, IDIM, HID), lambda b, be_ref: (be_ref[b], 0, 0)),
            ],
            out_specs=pl.BlockSpec((BLK, IDIM), lambda b, be_ref: (b, 0)),
        ),
        out_shape=jax.ShapeDtypeStruct((NROWS, IDIM), jnp.float32),
        compiler_params=pltpu.CompilerParams(
            dimension_semantics=("arbitrary",),
            vmem_limit_bytes=100 * 1024 * 1024,
        ),
    )(be, xsorted, w1b, w2b)


def _combine_body(ys_hbm, d0_hbm, d1_hbm, w0_hbm, w1_hbm, out_hbm,
                  y0v, y1v, ov, i0, i1, wv0, wv1, sem):
    wid = lax.axis_index("s") * 2 + lax.axis_index("c")
    base = wid * CHUNK
    lane = lax.iota(jnp.int32, 16)

    def step(j, _):
        off = base + j * CSUB
        pltpu.sync_copy(d0_hbm.at[pl.ds(off, CSUB)], i0)
        pltpu.sync_copy(d1_hbm.at[pl.ds(off, CSUB)], i1)
        pltpu.sync_copy(w0_hbm.at[pl.ds(off, CSUB)], wv0)
        pltpu.sync_copy(w1_hbm.at[pl.ds(off, CSUB)], wv1)
        pltpu.async_copy(ys_hbm.at[i0], y0v, sem).wait()
        pltpu.async_copy(ys_hbm.at[i1], y1v, sem).wait()
        wa = wv0[...]
        wb = wv1[...]

        def tok(t, _):
            a = jnp.broadcast_to(jnp.sum(jnp.where(lane == t, wa, 0.0)), (16,))
            b = jnp.broadcast_to(jnp.sum(jnp.where(lane == t, wb, 0.0)), (16,))

            def colgrp(g, _):
                for u in range(8):
                    sl = pl.ds(g * 128 + u * 16, 16)
                    ov[t, sl] = a * y0v[t, sl] + b * y1v[t, sl]
                return 0

            lax.fori_loop(0, IDIM // 128, colgrp, 0)
            return 0

        lax.fori_loop(0, CSUB, tok, 0)
        pltpu.sync_copy(ov, out_hbm.at[pl.ds(off, CSUB)])
        return 0

    lax.fori_loop(0, CHUNK // CSUB, step, 0)


def _combine(ysorted, d0, d1, wt0, wt1):
    f = pl.kernel(
        _combine_body,
        out_type=jax.ShapeDtypeStruct((T, IDIM), jnp.float32),
        mesh=_sc_mesh(),
        scratch_types=[
            pltpu.VMEM((CSUB, IDIM), jnp.float32),
            pltpu.VMEM((CSUB, IDIM), jnp.float32),
            pltpu.VMEM((CSUB, IDIM), jnp.float32),
            pltpu.VMEM((CSUB,), jnp.int32),
            pltpu.VMEM((CSUB,), jnp.int32),
            pltpu.VMEM((CSUB,), jnp.float32),
            pltpu.VMEM((CSUB,), jnp.float32),
            pltpu.SemaphoreType.DMA,
        ],
        compiler_params=pltpu.CompilerParams(needs_layout_passes=False),
    )
    return f(ysorted, d0, d1, wt0, wt1)


def kernel(xs, gate_w, w1, w2):
    x = xs.reshape(-1, IDIM)
    logits = _router(x, gate_w)
    d0, d1, wt0, wt1, ps2 = _route_meta(logits)
    ps = ps2[0]
    starts = jnp.arange(NBLK, dtype=jnp.int32) * BLK
    be = jnp.sum((starts[:, None] >= ps[None, :]).astype(jnp.int32), axis=1) - 1
    xsorted = _dispatch(x, d0, d1)
    ysorted = _gmm(be, xsorted, w1.astype(jnp.bfloat16), w2.astype(jnp.bfloat16))
    out = _combine(ysorted, d0, d1, wt0, wt1)
    return out.reshape(xs.shape)
